# baseline (speedup 1.0000x reference)
import functools

import jax
import jax.numpy as jnp
from jax import lax
from jax.experimental import pallas as pl
from jax.experimental.pallas import tpu as pltpu
from jax.experimental.pallas import tpu_sc as plsc

BLK = 128
SCORE_BLK = 2048
CP = 128


def _score_route_body(x_ref, w_ref, b_ref, xp_ref, dest_ref, eid_ref,
                      s2d_ref):
    bidx = pl.program_id(0)
    nsteps = pl.num_programs(0)
    xv = x_ref[...]
    st = lax.dot_general(
        w_ref[...], xv, (((0,), (1,)), ((), ())),
        preferred_element_type=jnp.float32,
    )
    rows_per_step = SCORE_BLK // 128
    s2d_ref[pl.ds(bidx * rows_per_step, rows_per_step)] = (
        st.reshape(rows_per_step, 128) + b_ref[0]
    )

    d = xv.shape[1]
    xb = xv.astype(jnp.bfloat16)
    lo = pltpu.bitcast(xb[:, : d // 2], jnp.uint16).astype(jnp.uint32)
    hi = pltpu.bitcast(xb[:, d // 2 :], jnp.uint16).astype(jnp.uint32)
    xp_ref[...] = pltpu.bitcast(lo | (hi << 16), jnp.int32)

    @pl.when(bidx == nsteps - 1)
    def _route():
        rows, lanes = s2d_ref.shape
        pred = s2d_ref[...] > 0.0
        t = pred.astype(jnp.float32)
        ju = lax.broadcasted_iota(jnp.int32, (lanes, lanes), 0)
        ku = lax.broadcasted_iota(jnp.int32, (lanes, lanes), 1)
        su = (ju < ku).astype(jnp.float32)
        ones = jnp.ones((lanes, lanes), jnp.float32)
        excl_row = jnp.dot(t, su, preferred_element_type=jnp.float32)
        rowsum = jnp.dot(t, ones, preferred_element_type=jnp.float32)
        jl = lax.broadcasted_iota(jnp.int32, (rows, rows), 0)
        kl = lax.broadcasted_iota(jnp.int32, (rows, rows), 1)
        slo = (kl < jl).astype(jnp.float32)
        offs = jnp.dot(slo, rowsum, preferred_element_type=jnp.float32)
        excl = excl_row + offs
        n_ap = jnp.sum(t)
        a = jnp.ceil(n_ap / BLK) * BLK
        gi = (
            lax.broadcasted_iota(jnp.int32, (rows, lanes), 0) * lanes
            + lax.broadcasted_iota(jnp.int32, (rows, lanes), 1)
        ).astype(jnp.float32)
        dest = jnp.where(pred, excl, a + gi - excl)
        dest_ref[...] = dest.astype(jnp.int32)
        blk = lax.broadcasted_iota(jnp.int32, eid_ref.shape, 1).astype(
            jnp.float32
        )
        eid_ref[...] = (blk * BLK >= a).astype(jnp.int32)


def _scores_route(x, W_cls, b_cls, nb):
    n, d = x.shape
    return pl.pallas_call(
        _score_route_body,
        grid=(n // SCORE_BLK,),
        in_specs=[
            pl.BlockSpec((SCORE_BLK, d), lambda b: (b, 0)),
            pl.BlockSpec((d, 1), lambda b: (0, 0)),
            pl.BlockSpec(memory_space=pltpu.SMEM),
        ],
        out_specs=(
            pl.BlockSpec((SCORE_BLK, d // 2), lambda b: (b, 0)),
            pl.BlockSpec((n // 128, 128), lambda b: (0, 0)),
            pl.BlockSpec((1, nb), lambda b: (0, 0)),
        ),
        out_shape=(
            jax.ShapeDtypeStruct((n, d // 2), jnp.int32),
            jax.ShapeDtypeStruct((n // 128, 128), jnp.int32),
            jax.ShapeDtypeStruct((1, nb), jnp.int32),
        ),
        scratch_shapes=[pltpu.VMEM((n // 128, 128), jnp.float32)],
    )(x, W_cls, b_cls)


def _dispatch(xp, dest2d, npad):
    n, dp = xp.shape
    info = plsc.get_sparse_core_info()
    nw = info.num_cores * info.num_subcores
    per_w = n // nw
    ch = 128
    nch = per_w // ch
    mesh = plsc.VectorSubcoreMesh(core_axis_name="c", subcore_axis_name="s")

    @functools.partial(
        pl.kernel,
        mesh=mesh,
        out_type=jax.ShapeDtypeStruct((npad, dp), jnp.int32),
        scratch_types=[
            pltpu.VMEM((ch, dp), jnp.int32),
            pltpu.VMEM((nch, ch), jnp.int32),
            pltpu.SemaphoreType.DMA,
        ],
    )
    def k(x_hbm, dest_hbm, xp_hbm, rows_v, idx_v, sem):
        wid = lax.axis_index("s") * info.num_cores + lax.axis_index("c")
        base = wid * per_w
        pltpu.sync_copy(dest_hbm.at[pl.ds(wid * nch, nch)], idx_v)
        for c in range(nch):
            pltpu.sync_copy(x_hbm.at[pl.ds(base + c * ch, ch)], rows_v)
            pltpu.async_copy(rows_v, xp_hbm.at[idx_v.at[c]], sem).wait()

    return k(xp, dest2d)


def _mlp_body(eid_ref, x_ref, wa1_ref, wp1_ref, b1a_ref, b1p_ref, wa2_ref,
              wp2_ref, b2a_ref, b2p_ref, o_ref, w1_ref, w2_ref):
    b = pl.program_id(0)

    @pl.when(b == 0)
    def _cast_weights():
        w1_ref[0] = wa1_ref[...].astype(jnp.bfloat16)
        w1_ref[1] = wp1_ref[...].astype(jnp.bfloat16)
        cc = wa2_ref.shape[1]
        w2_ref[...] = jnp.zeros_like(w2_ref)
        w2_ref[0, :, :cc] = wa2_ref[...].astype(jnp.bfloat16)
        w2_ref[1, :, :cc] = wp2_ref[...].astype(jnp.bfloat16)

    e = eid_ref[0, b]
    is_ap = e == 0
    pu = pltpu.bitcast(x_ref[...], jnp.uint32)
    lo = pltpu.bitcast((pu & 0xFFFF).astype(jnp.uint16), jnp.bfloat16)
    hi = pltpu.bitcast((pu >> 16).astype(jnp.uint16), jnp.bfloat16)
    xv = jnp.concatenate([lo, hi], axis=1)
    b1 = jnp.where(is_ap, b1a_ref[...], b1p_ref[...])
    h = jnp.maximum(
        jnp.dot(xv, w1_ref[e], preferred_element_type=jnp.float32) + b1,
        0.0,
    )
    c = b2a_ref.shape[1]
    b2c = jnp.where(is_ap, b2a_ref[...], b2p_ref[...])
    b2 = jnp.concatenate(
        [b2c, jnp.zeros((1, CP - c), jnp.float32)], axis=1
    )
    o_ref[...] = (
        jnp.dot(h.astype(jnp.bfloat16), w2_ref[e],
                preferred_element_type=jnp.float32)
        + b2
    )


def _mlp(eid2d, x_perm, W_ap1, W_pa1, b_ap1, b_pa1, W_ap2, W_pa2, b_ap2,
         b_pa2):
    npad, dp = x_perm.shape
    d = dp * 2
    f = W_ap1.shape[1]
    c = W_ap2.shape[1]
    nb = npad // BLK
    grid_spec = pltpu.PrefetchScalarGridSpec(
        num_scalar_prefetch=1,
        grid=(nb,),
        in_specs=[
            pl.BlockSpec((BLK, dp), lambda b, e: (b, 0)),
            pl.BlockSpec((d, f), lambda b, e: (0, 0)),
            pl.BlockSpec((d, f), lambda b, e: (0, 0)),
            pl.BlockSpec((1, f), lambda b, e: (0, 0)),
            pl.BlockSpec((1, f), lambda b, e: (0, 0)),
            pl.BlockSpec((f, c), lambda b, e: (0, 0)),
            pl.BlockSpec((f, c), lambda b, e: (0, 0)),
            pl.BlockSpec((1, c), lambda b, e: (0, 0)),
            pl.BlockSpec((1, c), lambda b, e: (0, 0)),
        ],
        out_specs=pl.BlockSpec((BLK, CP), lambda b, e: (b, 0)),
        scratch_shapes=[
            pltpu.VMEM((2, d, f), jnp.bfloat16),
            pltpu.VMEM((2, f, CP), jnp.bfloat16),
        ],
    )
    return pl.pallas_call(
        _mlp_body,
        grid_spec=grid_spec,
        out_shape=jax.ShapeDtypeStruct((npad, CP), jnp.float32),
        compiler_params=pltpu.CompilerParams(
            dimension_semantics=("arbitrary",)
        ),
    )(eid2d, x_perm, W_ap1, W_pa1, b_ap1.reshape(1, f), b_pa1.reshape(1, f),
      W_ap2, W_pa2, b_ap2.reshape(1, c), b_pa2.reshape(1, c))


def _combine(out_perm, dest2d, n):
    info = plsc.get_sparse_core_info()
    nw = info.num_cores * info.num_subcores
    per_w = n // nw
    mesh = plsc.VectorSubcoreMesh(core_axis_name="c", subcore_axis_name="s")

    @functools.partial(
        pl.kernel,
        mesh=mesh,
        out_type=jax.ShapeDtypeStruct((n, CP), jnp.float32),
        scratch_types=[
            pltpu.VMEM((per_w, CP), jnp.float32),
            pltpu.VMEM((per_w // 128, 128), jnp.int32),
            pltpu.SemaphoreType.DMA,
        ],
    )
    def k(op_hbm, dest_hbm, out_hbm, rows_v, idx_v, sem):
        wid = lax.axis_index("s") * info.num_cores + lax.axis_index("c")
        base = wid * per_w
        pltpu.sync_copy(dest_hbm.at[pl.ds(wid * (per_w // 128), per_w // 128)],
                        idx_v)
        for c in range(per_w // 128):
            pltpu.async_copy(
                op_hbm.at[idx_v.at[c]],
                rows_v.at[pl.ds(c * 128, 128)], sem,
            ).wait()
        pltpu.sync_copy(rows_v, out_hbm.at[pl.ds(base, per_w)])

    return k(out_perm, dest2d)


def kernel(x, W_cls, b_cls, W_ap1, b_ap1, W_ap2, b_ap2, W_pa1, b_pa1, W_pa2, b_pa2):
    n, d = x.shape
    f = W_ap1.shape[1]
    c = W_ap2.shape[1]
    nb = n // BLK + 1
    npad = nb * BLK

    xp, dest2d, eid2d = _scores_route(x, W_cls, b_cls, nb)
    x_perm = _dispatch(xp, dest2d, npad)
    out_perm = _mlp(eid2d, x_perm, W_ap1, W_pa1, b_ap1, b_pa1, W_ap2, W_pa2,
                    b_ap2, b_pa2)
    out = _combine(out_perm, dest2d, n)
    return out[:, :c]


# speedup vs baseline: 1.1101x; 1.1101x over previous
import functools

import jax
import jax.numpy as jnp
from jax import lax
from jax.experimental import pallas as pl
from jax.experimental.pallas import tpu as pltpu
from jax.experimental.pallas import tpu_sc as plsc

BLK = 256
SCORE_BLK = 2048
CP = 128


def _score_route_body(x_ref, w_ref, b_ref, xp_ref, dest_ref, eid_ref,
                      s2d_ref):
    bidx = pl.program_id(0)
    nsteps = pl.num_programs(0)
    xv = x_ref[...]
    st = lax.dot_general(
        w_ref[...], xv, (((0,), (1,)), ((), ())),
        preferred_element_type=jnp.float32,
    )
    rows_per_step = SCORE_BLK // 128
    s2d_ref[pl.ds(bidx * rows_per_step, rows_per_step)] = (
        st.reshape(rows_per_step, 128) + b_ref[0]
    )

    d = xv.shape[1]
    xb = xv.astype(jnp.bfloat16)
    lo = pltpu.bitcast(xb[:, : d // 2], jnp.uint16).astype(jnp.uint32)
    hi = pltpu.bitcast(xb[:, d // 2 :], jnp.uint16).astype(jnp.uint32)
    xp_ref[...] = pltpu.bitcast(lo | (hi << 16), jnp.int32)

    @pl.when(bidx == nsteps - 1)
    def _route():
        rows, lanes = s2d_ref.shape
        pred = s2d_ref[...] > 0.0
        t = pred.astype(jnp.float32)
        ju = lax.broadcasted_iota(jnp.int32, (lanes, lanes), 0)
        ku = lax.broadcasted_iota(jnp.int32, (lanes, lanes), 1)
        su = (ju < ku).astype(jnp.float32)
        ones = jnp.ones((lanes, lanes), jnp.float32)
        excl_row = jnp.dot(t, su, preferred_element_type=jnp.float32)
        rowsum = jnp.dot(t, ones, preferred_element_type=jnp.float32)
        jl = lax.broadcasted_iota(jnp.int32, (rows, rows), 0)
        kl = lax.broadcasted_iota(jnp.int32, (rows, rows), 1)
        slo = (kl < jl).astype(jnp.float32)
        offs = jnp.dot(slo, rowsum, preferred_element_type=jnp.float32)
        excl = excl_row + offs
        n_ap = jnp.sum(t)
        a = jnp.ceil(n_ap / BLK) * BLK
        gi = (
            lax.broadcasted_iota(jnp.int32, (rows, lanes), 0) * lanes
            + lax.broadcasted_iota(jnp.int32, (rows, lanes), 1)
        ).astype(jnp.float32)
        dest = jnp.where(pred, excl, a + gi - excl)
        dest_ref[...] = dest.astype(jnp.int32)
        blk = lax.broadcasted_iota(jnp.int32, eid_ref.shape, 1).astype(
            jnp.float32
        )
        eid_ref[...] = (blk * BLK >= a).astype(jnp.int32)


def _scores_route(x, W_cls, b_cls, nb):
    n, d = x.shape
    return pl.pallas_call(
        _score_route_body,
        grid=(n // SCORE_BLK,),
        in_specs=[
            pl.BlockSpec((SCORE_BLK, d), lambda b: (b, 0)),
            pl.BlockSpec((d, 1), lambda b: (0, 0)),
            pl.BlockSpec(memory_space=pltpu.SMEM),
        ],
        out_specs=(
            pl.BlockSpec((SCORE_BLK, d // 2), lambda b: (b, 0)),
            pl.BlockSpec((n // 128, 128), lambda b: (0, 0)),
            pl.BlockSpec((1, nb), lambda b: (0, 0)),
        ),
        out_shape=(
            jax.ShapeDtypeStruct((n, d // 2), jnp.int32),
            jax.ShapeDtypeStruct((n // 128, 128), jnp.int32),
            jax.ShapeDtypeStruct((1, nb), jnp.int32),
        ),
        scratch_shapes=[pltpu.VMEM((n // 128, 128), jnp.float32)],
    )(x, W_cls, b_cls)


def _dispatch(xp, dest2d, npad):
    n, dp = xp.shape
    info = plsc.get_sparse_core_info()
    nw = info.num_cores * info.num_subcores
    per_w = n // nw
    ch = 128
    nch = per_w // ch
    mesh = plsc.VectorSubcoreMesh(core_axis_name="c", subcore_axis_name="s")

    @functools.partial(
        pl.kernel,
        mesh=mesh,
        out_type=jax.ShapeDtypeStruct((npad, dp), jnp.int32),
        scratch_types=[
            pltpu.VMEM((ch, dp), jnp.int32),
            pltpu.VMEM((nch, ch), jnp.int32),
            pltpu.SemaphoreType.DMA,
        ],
    )
    def k(x_hbm, dest_hbm, xp_hbm, rows_v, idx_v, sem):
        wid = lax.axis_index("s") * info.num_cores + lax.axis_index("c")
        base = wid * per_w
        pltpu.sync_copy(dest_hbm.at[pl.ds(wid * nch, nch)], idx_v)
        for c in range(nch):
            pltpu.sync_copy(x_hbm.at[pl.ds(base + c * ch, ch)], rows_v)
            pltpu.async_copy(rows_v, xp_hbm.at[idx_v.at[c]], sem).wait()

    return k(xp, dest2d)


def _mlp_body(eid_ref, x_ref, wa1_ref, wp1_ref, b1a_ref, b1p_ref, wa2_ref,
              wp2_ref, b2a_ref, b2p_ref, o_ref, w1_ref, w2_ref):
    b = pl.program_id(0)

    @pl.when(b == 0)
    def _cast_weights():
        w1_ref[0] = wa1_ref[...].astype(jnp.bfloat16)
        w1_ref[1] = wp1_ref[...].astype(jnp.bfloat16)
        cc = wa2_ref.shape[1]
        w2_ref[...] = jnp.zeros_like(w2_ref)
        w2_ref[0, :, :cc] = wa2_ref[...].astype(jnp.bfloat16)
        w2_ref[1, :, :cc] = wp2_ref[...].astype(jnp.bfloat16)

    e = eid_ref[0, b]
    is_ap = e == 0
    pu = pltpu.bitcast(x_ref[...], jnp.uint32)
    lo = pltpu.bitcast((pu & 0xFFFF).astype(jnp.uint16), jnp.bfloat16)
    hi = pltpu.bitcast((pu >> 16).astype(jnp.uint16), jnp.bfloat16)
    xv = jnp.concatenate([lo, hi], axis=1)
    b1 = jnp.where(is_ap, b1a_ref[...], b1p_ref[...])
    h = jnp.maximum(
        jnp.dot(xv, w1_ref[e], preferred_element_type=jnp.float32) + b1,
        0.0,
    )
    c = b2a_ref.shape[1]
    b2c = jnp.where(is_ap, b2a_ref[...], b2p_ref[...])
    b2 = jnp.concatenate(
        [b2c, jnp.zeros((1, CP - c), jnp.float32)], axis=1
    )
    o_ref[...] = (
        jnp.dot(h.astype(jnp.bfloat16), w2_ref[e],
                preferred_element_type=jnp.float32)
        + b2
    )


def _mlp(eid2d, x_perm, W_ap1, W_pa1, b_ap1, b_pa1, W_ap2, W_pa2, b_ap2,
         b_pa2):
    npad, dp = x_perm.shape
    d = dp * 2
    f = W_ap1.shape[1]
    c = W_ap2.shape[1]
    nb = npad // BLK
    grid_spec = pltpu.PrefetchScalarGridSpec(
        num_scalar_prefetch=1,
        grid=(nb,),
        in_specs=[
            pl.BlockSpec((BLK, dp), lambda b, e: (b, 0)),
            pl.BlockSpec((d, f), lambda b, e: (0, 0)),
            pl.BlockSpec((d, f), lambda b, e: (0, 0)),
            pl.BlockSpec((1, f), lambda b, e: (0, 0)),
            pl.BlockSpec((1, f), lambda b, e: (0, 0)),
            pl.BlockSpec((f, c), lambda b, e: (0, 0)),
            pl.BlockSpec((f, c), lambda b, e: (0, 0)),
            pl.BlockSpec((1, c), lambda b, e: (0, 0)),
            pl.BlockSpec((1, c), lambda b, e: (0, 0)),
        ],
        out_specs=pl.BlockSpec((BLK, CP), lambda b, e: (b, 0)),
        scratch_shapes=[
            pltpu.VMEM((2, d, f), jnp.bfloat16),
            pltpu.VMEM((2, f, CP), jnp.bfloat16),
        ],
    )
    return pl.pallas_call(
        _mlp_body,
        grid_spec=grid_spec,
        out_shape=jax.ShapeDtypeStruct((npad, CP), jnp.float32),
        compiler_params=pltpu.CompilerParams(
            dimension_semantics=("arbitrary",)
        ),
    )(eid2d, x_perm, W_ap1, W_pa1, b_ap1.reshape(1, f), b_pa1.reshape(1, f),
      W_ap2, W_pa2, b_ap2.reshape(1, c), b_pa2.reshape(1, c))


def _combine(out_perm, dest2d, n):
    info = plsc.get_sparse_core_info()
    nw = info.num_cores * info.num_subcores
    per_w = n // nw
    mesh = plsc.VectorSubcoreMesh(core_axis_name="c", subcore_axis_name="s")

    @functools.partial(
        pl.kernel,
        mesh=mesh,
        out_type=jax.ShapeDtypeStruct((n, CP), jnp.float32),
        scratch_types=[
            pltpu.VMEM((per_w, CP), jnp.float32),
            pltpu.VMEM((per_w // 128, 128), jnp.int32),
            pltpu.SemaphoreType.DMA,
        ],
    )
    def k(op_hbm, dest_hbm, out_hbm, rows_v, idx_v, sem):
        wid = lax.axis_index("s") * info.num_cores + lax.axis_index("c")
        base = wid * per_w
        pltpu.sync_copy(dest_hbm.at[pl.ds(wid * (per_w // 128), per_w // 128)],
                        idx_v)
        for c in range(per_w // 128):
            pltpu.async_copy(
                op_hbm.at[idx_v.at[c]],
                rows_v.at[pl.ds(c * 128, 128)], sem,
            ).wait()
        pltpu.sync_copy(rows_v, out_hbm.at[pl.ds(base, per_w)])

    return k(out_perm, dest2d)


def kernel(x, W_cls, b_cls, W_ap1, b_ap1, W_ap2, b_ap2, W_pa1, b_pa1, W_pa2, b_pa2):
    n, d = x.shape
    f = W_ap1.shape[1]
    c = W_ap2.shape[1]
    nb = n // BLK + 1
    npad = nb * BLK

    xp, dest2d, eid2d = _scores_route(x, W_cls, b_cls, nb)
    x_perm = _dispatch(xp, dest2d, npad)
    out_perm = _mlp(eid2d, x_perm, W_ap1, W_pa1, b_ap1, b_pa1, W_ap2, W_pa2,
                    b_ap2, b_pa2)
    out = _combine(out_perm, dest2d, n)
    return out[:, :c]
